# Initial kernel scaffold; baseline (speedup 1.0000x reference)
import jax
import jax.numpy as jnp
from jax.experimental import pallas as pl


def kernel(x, edge_index, W1, b1, W2, b2, gamma1, beta1, gamma2, beta2, Aw1, Ab1, Aw2, Ab2, We, be, gammae, betae, Wb, bb, gammab, betab):
    raise NotImplementedError("write your pallas kernel here")


# speedup vs baseline: 17.0040x; 17.0040x over previous
import functools

import jax
import jax.numpy as jnp
from jax import lax
from jax.experimental import pallas as pl
from jax.experimental.pallas import tpu as pltpu
from jax.experimental.pallas import tpu_sc as plsc

N = 10000
E = 320000
NPAD = 10240
H = 128
BOT = 32
NC = 2
NS = 16
NW = NC * NS
EPT = E // NW
K = 80
G = EPT // K
RPT = NPAD // NS
BNS = (1.0 + 1e-5) ** -0.5
PREC = lax.Precision.HIGHEST

_mesh = plsc.VectorSubcoreMesh(
    core_axis_name="c", subcore_axis_name="s", num_cores=NC, num_subcores=NS
)


def _deg_body(dst_hbm, hist_hbm, dstv, hist):
    c = lax.axis_index("c")
    s = lax.axis_index("s")
    wid = c * NS + s

    def zero_body(i, _):
        hist[pl.ds(i * 16, 16)] = jnp.zeros((16,), jnp.float32)
        return 0

    lax.fori_loop(0, NPAD // 16, zero_body, 0)
    pltpu.sync_copy(dst_hbm.at[wid], dstv)
    ones16 = jnp.ones((16,), jnp.float32)

    def count_body(i, _):
        idx16 = dstv[pl.ds(i * 16, 16)]
        plsc.addupdate_scatter(hist, [idx16], ones16)
        return 0

    lax.fori_loop(0, EPT // 16, count_body, 0)
    pltpu.sync_copy(hist, hist_hbm.at[wid])


_deg_kernel = pl.kernel(
    _deg_body,
    out_type=jax.ShapeDtypeStruct((NW, NPAD), jnp.float32),
    mesh=_mesh,
    compiler_params=pltpu.CompilerParams(needs_layout_passes=False),
    scratch_types=[
        pltpu.VMEM((EPT,), jnp.int32),
        pltpu.VMEM((NPAD,), jnp.float32),
    ],
)


def _spmm_body(table_hbm, src_hbm, dst_hbm, acc_hbm, srcv, dstv, rows, accsp, sem):
    c = lax.axis_index("c")
    s = lax.axis_index("s")
    wid = c * NS + s

    def zrow(i, _):
        r = i // 8
        k = i % 8
        rows[r, pl.ds(k * 16, 16)] = jnp.zeros((16,), jnp.float32)
        return 0

    lax.fori_loop(0, K * 8, zrow, 0)
    for b in range(RPT // K):
        pltpu.sync_copy(rows, accsp.at[pl.ds(s * RPT + b * K, K)])
    plsc.subcore_barrier()

    pltpu.sync_copy(src_hbm.at[wid], srcv)
    pltpu.sync_copy(dst_hbm.at[wid], dstv)

    def chunk(g, _):
        pltpu.async_copy(table_hbm.at[srcv.at[g]], rows, sem).wait()
        pltpu.sync_copy(rows, accsp.at[dstv.at[g]], add=True)
        return 0

    lax.fori_loop(0, G, chunk, 0)
    plsc.subcore_barrier()
    for b in range(RPT // K):
        pltpu.sync_copy(
            accsp.at[pl.ds(s * RPT + b * K, K)],
            acc_hbm.at[c, pl.ds(s * RPT + b * K, K)],
        )


_spmm_kernel = pl.kernel(
    _spmm_body,
    out_type=jax.ShapeDtypeStruct((NC, NPAD, H), jnp.float32),
    mesh=_mesh,
    compiler_params=pltpu.CompilerParams(needs_layout_passes=False),
    scratch_types=[
        pltpu.VMEM((G, K), jnp.int32),
        pltpu.VMEM((G, K), jnp.int32),
        pltpu.VMEM((K, H), jnp.float32),
        pltpu.VMEM_SHARED((NPAD, H), jnp.float32),
        pltpu.SemaphoreType.DMA,
    ],
)


def _pre_body(x_ref, hist_ref, w1_ref, h1p_ref, dis_ref):
    ones = jnp.ones((NW, 1), jnp.float32)
    deg = 1.0 + lax.dot_general(
        hist_ref[...], ones, (((0,), (0,)), ((), ())), precision=PREC
    )
    dis = lax.rsqrt(deg)
    dis_ref[...] = dis
    h1p_ref[...] = jnp.dot(x_ref[...], w1_ref[...], precision=PREC) * dis


def _pre_stage(x_pad, hist, w1):
    blk = 1024
    grid = NPAD // blk
    return pl.pallas_call(
        _pre_body,
        grid=(grid,),
        in_specs=[
            pl.BlockSpec((blk, H), lambda i: (i, 0)),
            pl.BlockSpec((NW, blk), lambda i: (0, i)),
            pl.BlockSpec((H, H), lambda i: (0, 0)),
        ],
        out_specs=[
            pl.BlockSpec((blk, H), lambda i: (i, 0)),
            pl.BlockSpec((blk, 1), lambda i: (i, 0)),
        ],
        out_shape=[
            jax.ShapeDtypeStruct((NPAD, H), jnp.float32),
            jax.ShapeDtypeStruct((NPAD, 1), jnp.float32),
        ],
    )(x_pad, hist, w1)


def _mid_body(acc_ref, h1p_ref, dis_ref, b1_ref, g1_ref, be1_ref, w2_ref, h2p_ref):
    dis = dis_ref[...]
    g = (acc_ref[0, :, :] + acc_ref[1, :, :] + h1p_ref[...]) * dis
    pre = (g + b1_ref[...]) * BNS * g1_ref[...] + be1_ref[...]
    out1 = jnp.maximum(pre, 0.0)
    h2p_ref[...] = jnp.dot(out1, w2_ref[...], precision=PREC) * dis


def _mid_stage(acc, h1p, dis, b1, gamma1, beta1, w2):
    blk = 1024
    grid = NPAD // blk
    return pl.pallas_call(
        _mid_body,
        grid=(grid,),
        in_specs=[
            pl.BlockSpec((NC, blk, H), lambda i: (0, i, 0)),
            pl.BlockSpec((blk, H), lambda i: (i, 0)),
            pl.BlockSpec((blk, 1), lambda i: (i, 0)),
            pl.BlockSpec((H,), lambda i: (0,)),
            pl.BlockSpec((H,), lambda i: (0,)),
            pl.BlockSpec((H,), lambda i: (0,)),
            pl.BlockSpec((H, H), lambda i: (0, 0)),
        ],
        out_specs=pl.BlockSpec((blk, H), lambda i: (i, 0)),
        out_shape=jax.ShapeDtypeStruct((NPAD, H), jnp.float32),
    )(acc, h1p, dis, b1, gamma1, beta1, w2)


def _emb_body(acc_ref, h2p_ref, dis_ref, b2_ref, g2_ref, be2_ref, emb_ref):
    g = (acc_ref[0, :, :] + acc_ref[1, :, :] + h2p_ref[...]) * dis_ref[...]
    pre = (g + b2_ref[...]) * BNS * g2_ref[...] + be2_ref[...]
    emb_ref[...] = jnp.maximum(pre, 0.0)


def _emb_stage(acc, h2p, dis, b2, gamma2, beta2):
    blk = 1024
    grid = NPAD // blk
    return pl.pallas_call(
        _emb_body,
        grid=(grid,),
        in_specs=[
            pl.BlockSpec((NC, blk, H), lambda i: (0, i, 0)),
            pl.BlockSpec((blk, H), lambda i: (i, 0)),
            pl.BlockSpec((blk, 1), lambda i: (i, 0)),
            pl.BlockSpec((H,), lambda i: (0,)),
            pl.BlockSpec((H,), lambda i: (0,)),
            pl.BlockSpec((H,), lambda i: (0,)),
        ],
        out_specs=pl.BlockSpec((blk, H), lambda i: (i, 0)),
        out_shape=jax.ShapeDtypeStruct((NPAD, H), jnp.float32),
    )(acc, h2p, dis, b2, gamma2, beta2)


def _pool_body(xr_ref, aw1_ref, ab1_ref, aw2_ref, pooled_ref):
    x2 = xr_ref[...].reshape(100, H)
    t = jnp.tanh(jnp.dot(x2, aw1_ref[...], precision=PREC) + ab1_ref[...])
    sc = jnp.dot(t, aw2_ref[...], precision=PREC)
    m = jnp.max(sc, axis=0, keepdims=True)
    e = jnp.exp(sc - m)
    w = e / jnp.sum(e, axis=0, keepdims=True)
    pooled_ref[...] = jnp.sum(x2 * w, axis=0).reshape(1, 1, H)


def _pool_stage(xr, aw1, ab1, aw2):
    return pl.pallas_call(
        _pool_body,
        grid=(100,),
        in_specs=[
            pl.BlockSpec((1, 100, H), lambda i: (i, 0, 0)),
            pl.BlockSpec((H, H // 2), lambda i: (0, 0)),
            pl.BlockSpec((H // 2,), lambda i: (0,)),
            pl.BlockSpec((H // 2, 1), lambda i: (0, 0)),
        ],
        out_specs=pl.BlockSpec((1, 1, H), lambda i: (i, 0, 0)),
        out_shape=jax.ShapeDtypeStruct((100, 1, H), jnp.float32),
    )(xr, aw1, ab1, aw2)


def _head_body(p_ref, we_ref, be_ref, ge_ref, bee_ref, wb_ref, bb_ref, gb_ref,
               beb_ref, out_ref):
    e = (jnp.dot(p_ref[...], we_ref[...], precision=PREC) + be_ref[...])
    e = jnp.maximum(e * BNS * ge_ref[...] + bee_ref[...], 0.0)
    o = (jnp.dot(e, wb_ref[...], precision=PREC) + bb_ref[...])
    out_ref[...] = jnp.maximum(o * BNS * gb_ref[...] + beb_ref[...], 0.0)


def _head_stage(pooled, we, be, gammae, betae, wb, bb, gammab, betab):
    return pl.pallas_call(
        _head_body,
        out_shape=jax.ShapeDtypeStruct((100, BOT), jnp.float32),
    )(pooled, we, be, gammae, betae, wb, bb, gammab, betab)


@jax.jit
def kernel(x, edge_index, W1, b1, W2, b2, gamma1, beta1, gamma2, beta2,
           Aw1, Ab1, Aw2, Ab2, We, be, gammae, betae, Wb, bb, gammab, betab):
    src_r = edge_index[0].reshape(NW, G, K)
    dst_r = edge_index[1].reshape(NW, G, K)
    dst_flat = edge_index[1].reshape(NW, EPT)
    x_pad = jnp.pad(x, ((0, NPAD - N), (0, 0)))

    hist = _deg_kernel(dst_flat)
    h1p, dis = _pre_stage(x_pad, hist, W1)
    acc1 = _spmm_kernel(h1p, src_r, dst_r)
    h2p = _mid_stage(acc1, h1p, dis, b1, gamma1, beta1, W2)
    acc2 = _spmm_kernel(h2p, src_r, dst_r)
    emb_pad = _emb_stage(acc2, h2p, dis, b2, gamma2, beta2)
    emb = emb_pad[:N]
    xr = emb.reshape(100, 100, H)
    pooled = _pool_stage(xr, Aw1, Ab1, Aw2).reshape(100, H)
    out = _head_stage(pooled, We, be, gammae, betae, Wb, bb, gammab, betab)
    return (out, emb)
